# baseline (speedup 1.0000x reference)
import jax
import jax.numpy as jnp
import numpy as np
from jax import lax
from jax.experimental import pallas as pl
from jax.experimental.pallas import tpu as pltpu

_NUM_CODES = 128
_CODE_DIM = 48
_COMMIT_W = 0.25
_EPS = 1e-05
_ENT_W = 0.1
_ENT_LO = 0.5
_ENT_HI = 0.9
_VAR_FLOOR = 0.05
_VAR_W = 0.001
_DECOR_W = 0.001

_N_ROWS = 128 * 1024
_BLK = 8192
_N_BLOCKS = _N_ROWS // _BLK


def _tc_body(x_ref, w_ref, u_ref, idx_ref, q_ref, tot_ref, com_ref, ent_ref,
             var_ref, dec_ref, ue_ref, acc_ref):
    i = pl.program_id(0)
    x = x_ref[...].reshape(_BLK, _CODE_DIM)
    w = w_ref[...]

    xsq = jnp.sum(x * x, axis=1, keepdims=True)
    wsq = jnp.sum(w * w, axis=1)
    mm = jax.lax.dot_general(x, w, (((1,), (1,)), ((), ())),
                             preferred_element_type=jnp.float32)
    d = xsq - 2.0 * mm + wsq[None, :]

    dt = d.T
    mind = jnp.min(dt, axis=0, keepdims=True)
    code_iota = lax.broadcasted_iota(jnp.int32, dt.shape, 0)
    idx = jnp.min(jnp.where(dt == mind, code_iota, _NUM_CODES), axis=0)
    idx_ref[...] = idx

    onehot_t = (code_iota == idx[None, :]).astype(jnp.float32)
    q = jax.lax.dot_general(onehot_t, w, (((0,), (0,)), ((), ())),
                            preferred_element_type=jnp.float32)
    q_ref[...] = q.reshape(q_ref.shape)

    blk_sum = jnp.sum(mind)

    @pl.when(i == 0)
    def _init():
        acc_ref[0] = blk_sum

    @pl.when(i > 0)
    def _acc():
        acc_ref[0] = acc_ref[0] + blk_sum

    @pl.when(i == _N_BLOCKS - 1)
    def _finalize():
        commit = _COMMIT_W * acc_ref[0] / float(_N_ROWS * _CODE_DIM)

        u = u_ref[...]
        p = u + _EPS
        p = p / jnp.maximum(jnp.sum(p), _EPS * _NUM_CODES)
        entropy = -jnp.sum(p * jnp.log(p + _EPS))
        ue = entropy / np.log(float(_NUM_CODES))
        gap = jnp.where(ue < _ENT_LO, _ENT_LO - ue,
                        jnp.where(ue > _ENT_HI, ue - _ENT_HI, 0.0))
        ent_loss = _ENT_W * gap * gap

        mean_w = jnp.mean(w, axis=0, keepdims=True)
        wc = w - mean_w
        variance = jnp.mean(wc * wc, axis=0, keepdims=True)
        var_loss = _VAR_W * jnp.mean(jnp.maximum(_VAR_FLOOR - variance, 0.0))

        cov = jax.lax.dot_general(wc, wc, (((0,), (0,)), ((), ())),
                                  preferred_element_type=jnp.float32)
        cov = cov / float(_NUM_CODES)
        ii = lax.broadcasted_iota(jnp.int32, cov.shape, 0)
        jj = lax.broadcasted_iota(jnp.int32, cov.shape, 1)
        off = jnp.where(ii == jj, 0.0, cov)
        dec_loss = _DECOR_W * jnp.sum(off * off) / float(_CODE_DIM * _CODE_DIM)

        tot_ref[...] = jnp.reshape(commit + ent_loss + var_loss + dec_loss,
                                   (1, 1))
        com_ref[...] = jnp.reshape(commit, (1, 1))
        ent_ref[...] = jnp.reshape(ent_loss, (1, 1))
        var_ref[...] = jnp.reshape(var_loss, (1, 1))
        dec_ref[...] = jnp.reshape(dec_loss, (1, 1))
        ue_ref[...] = jnp.reshape(ue, (1, 1))


def _tc_search(flat3, w, u2, interpret=False):
    scal = jax.ShapeDtypeStruct((1, 1), jnp.float32)
    return pl.pallas_call(
        _tc_body,
        grid=(_N_BLOCKS,),
        in_specs=[
            pl.BlockSpec((1, _BLK, _CODE_DIM), lambda i: (i, 0, 0)),
            pl.BlockSpec((_NUM_CODES, _CODE_DIM), lambda i: (0, 0)),
            pl.BlockSpec((1, _NUM_CODES), lambda i: (0, 0)),
        ],
        out_specs=[
            pl.BlockSpec((_BLK,), lambda i: (i,)),
            pl.BlockSpec((1, _BLK, _CODE_DIM), lambda i: (i, 0, 0)),
            pl.BlockSpec((1, 1), lambda i: (0, 0)),
            pl.BlockSpec((1, 1), lambda i: (0, 0)),
            pl.BlockSpec((1, 1), lambda i: (0, 0)),
            pl.BlockSpec((1, 1), lambda i: (0, 0)),
            pl.BlockSpec((1, 1), lambda i: (0, 0)),
            pl.BlockSpec((1, 1), lambda i: (0, 0)),
        ],
        out_shape=[
            jax.ShapeDtypeStruct((_N_ROWS,), jnp.int32),
            jax.ShapeDtypeStruct((_N_BLOCKS, _BLK, _CODE_DIM), jnp.float32),
            scal, scal, scal, scal, scal, scal,
        ],
        scratch_shapes=[pltpu.SMEM((1,), jnp.float32)],
        compiler_params=pltpu.CompilerParams(
            allow_input_fusion=[True, False, False]),
        interpret=interpret,
    )(flat3, w, u2)


def kernel(inputs, W, usage_counts):
    flat3 = inputs.reshape(_N_BLOCKS, _BLK, _CODE_DIM)
    idx_flat, q3, tot, com, ent, var, dec, ue = _tc_search(
        flat3, W, usage_counts.reshape(1, _NUM_CODES))
    quantized = q3.reshape(inputs.shape)
    indices = idx_flat.reshape(inputs.shape[:-1])
    return (quantized, indices, tot.reshape(()), com.reshape(()),
            ent.reshape(()), var.reshape(()), dec.reshape(()),
            ue.reshape(()))
